# baseline (speedup 1.0000x reference)
import functools
import math

import jax
import jax.numpy as jnp
from jax import lax
from jax.experimental import pallas as pl
from jax.experimental.pallas import tpu as pltpu
from jax.experimental.pallas import tpu_sc as plsc

NI = 10000
NU = 10000
DD = 256
HH = 128
EE = 320000
ROWS = 10240
DUMMY = NI
NSUB = 16
SLAB = ROWS // NSUB
CH = 32
G = 64
NB = 8
C0 = 640
C1 = 320
BNC = 1.0 / math.sqrt(1.0 + 1e-5)
R = 400


def _zero_fill(buf):
    zeros = jnp.zeros((16,), jnp.float32)

    def row(r, carry):
        def col(k, c2):
            buf[r, pl.ds(k * 16, 16)] = zeros
            return c2
        return lax.fori_loop(0, HH // 16, col, carry)

    lax.fori_loop(0, CH, row, 0)


def _agg_run(tab, src_slab, dst_slab, sidx, didx, bufs, acc,
             gsems, ssems, ngroups):
    def group(g, carry):
        pltpu.sync_copy(src_slab.at[pl.ds(g * G, G)], sidx)
        pltpu.sync_copy(dst_slab.at[pl.ds(g * G, G)], didx)
        for j in range(NB):
            pltpu.async_copy(tab.at[sidx.at[j]], bufs[j], gsems[j])

        def wave(w, c2):
            for j in range(NB):
                c = w * NB + j
                pltpu.make_async_copy(tab.at[sidx.at[c]],
                                      bufs[j], gsems[j]).wait()
                pltpu.async_copy(bufs[j], acc.at[didx.at[c]], ssems[j],
                                 add=True)

                @pl.when(w < G // NB - 1)
                def _(c=c, j=j):
                    pltpu.make_async_copy(bufs[j], acc.at[didx.at[c]],
                                          ssems[j]).wait()
                    pltpu.async_copy(tab.at[sidx.at[c + NB]],
                                     bufs[j], gsems[j])

            return c2

        lax.fori_loop(0, G // NB, wave, carry)
        for j in range(NB):
            pltpu.make_async_copy(bufs[j], acc.at[didx.at[G - NB + j]],
                                  ssems[j]).wait()
        return carry

    lax.fori_loop(0, ngroups, group, 0)


def _acc_zero(buf, acc, sid):
    _zero_fill(buf)
    for j in range(SLAB // CH):
        pltpu.sync_copy(buf, acc.at[pl.ds(sid * SLAB + j * CH, CH)])


def _writeback(acc, out, sid):
    pltpu.sync_copy(acc.at[pl.ds(sid * SLAB, SLAB)],
                    out.at[pl.ds(sid * SLAB, SLAB)])


_SC_MESH = plsc.VectorSubcoreMesh(core_axis_name="c", subcore_axis_name="s")

_AGG_SCRATCH = (
    [pltpu.VMEM((G, CH), jnp.int32)] * 2
    + [pltpu.VMEM((CH, HH), jnp.float32)] * NB
    + [pltpu.VMEM_SHARED((ROWS, HH), jnp.float32)]
    + [pltpu.SemaphoreType.DMA] * (2 * NB)
)


@functools.partial(
    pl.kernel,
    mesh=_SC_MESH,
    out_type=[jax.ShapeDtypeStruct((ROWS, HH), jnp.float32)] * 4,
    scratch_types=_AGG_SCRATCH,
)
def _agg_layer0(xu_lo, xu_hi, xi_lo, xi_hi, ub_src, ub_dst, iu_src, iu_dst,
                oi_lo, oi_hi, ou_lo, ou_hi,
                sidx, didx, b0, b1, b2, b3, b4, b5, b6, b7, acc,
                g0, g1, g2, g3, g4, g5, g6, g7,
                s0, s1, s2, s3, s4, s5, s6, s7):
    cid = lax.axis_index("c")
    sid = lax.axis_index("s")
    bufs = [b0, b1, b2, b3, b4, b5, b6, b7]
    gsems = [g0, g1, g2, g3, g4, g5, g6, g7]
    ssems = [s0, s1, s2, s3, s4, s5, s6, s7]

    def one_pass(tab_lo, tab_hi, src, dst, out_lo, out_hi):
        _acc_zero(b0, acc, sid)
        plsc.subcore_barrier()

        @pl.when(cid == 0)
        def _():
            _agg_run(tab_lo, src.at[sid], dst.at[sid], sidx, didx,
                     bufs, acc, gsems, ssems, C0 // G)

        @pl.when(cid != 0)
        def _():
            _agg_run(tab_hi, src.at[sid], dst.at[sid], sidx, didx,
                     bufs, acc, gsems, ssems, C0 // G)

        plsc.subcore_barrier()

        @pl.when(cid == 0)
        def _():
            _writeback(acc, out_lo, sid)

        @pl.when(cid != 0)
        def _():
            _writeback(acc, out_hi, sid)

        plsc.subcore_barrier()

    one_pass(xu_lo, xu_hi, ub_src, ub_dst, oi_lo, oi_hi)
    one_pass(xi_lo, xi_hi, iu_src, iu_dst, ou_lo, ou_hi)


@functools.partial(
    pl.kernel,
    mesh=_SC_MESH,
    out_type=[jax.ShapeDtypeStruct((ROWS, HH), jnp.float32)] * 2,
    scratch_types=_AGG_SCRATCH,
)
def _agg_layer1(tab, src4, dst4, out_a, out_b, sidx, didx,
                b0, b1, b2, b3, b4, b5, b6, b7, acc,
                g0, g1, g2, g3, g4, g5, g6, g7,
                s0, s1, s2, s3, s4, s5, s6, s7):
    cid = lax.axis_index("c")
    sid = lax.axis_index("s")
    _acc_zero(b0, acc, sid)
    plsc.subcore_barrier()

    _agg_run(tab, src4.at[cid, sid], dst4.at[cid, sid], sidx, didx,
             [b0, b1, b2, b3, b4, b5, b6, b7], acc,
             [g0, g1, g2, g3, g4, g5, g6, g7],
             [s0, s1, s2, s3, s4, s5, s6, s7], C1 // G)

    plsc.subcore_barrier()

    @pl.when(cid == 0)
    def _():
        _writeback(acc, out_a, sid)

    @pl.when(cid != 0)
    def _():
        _writeback(acc, out_b, sid)


def _pad_idx(idx, total):
    pad = jnp.arange(total - EE, dtype=jnp.int32) % NI
    return jnp.concatenate([idx.astype(jnp.int32), pad])


def _pad_dst(idx, total):
    pad = jnp.arange(total - EE, dtype=jnp.int32) % (ROWS - NI) + DUMMY
    return jnp.concatenate([idx.astype(jnp.int32), pad])


def _stage_a_body(xi_ref, y_ref, emb_ref, xi0_ref):
    y = y_ref[...]
    e0 = emb_ref[0:1, :]
    e1 = emb_ref[1:2, :]
    sel0 = jnp.where(y == 0, 1.0, 0.0)
    sel1 = jnp.where(y == 1, 1.0, 0.0)
    xi0_ref[...] = xi_ref[...] + sel0 * e0 + sel1 * e1


def _stage_b_body(ai_lo_ref, ai_hi_ref, au_lo_ref, au_hi_ref, xi0_ref, xu_ref,
                  wrel_i_lo_ref, wrel_i_hi_ref, wrel_u_lo_ref, wrel_u_hi_ref,
                  wro_ub_ref, wro_iu_ref,
                  brel_i_ref, brel_u_ref, g0i_ref, b0i_ref, g0u_ref, b0u_ref,
                  xi1_ref, xu1_ref):
    dot = functools.partial(jnp.dot, preferred_element_type=jnp.float32)
    ni = (dot(ai_lo_ref[...], wrel_i_lo_ref[...])
          + dot(ai_hi_ref[...], wrel_i_hi_ref[...])
          + brel_i_ref[...] + dot(xi0_ref[...], wro_ub_ref[...]))
    xi1_ref[...] = jnp.maximum(g0i_ref[...] * (ni * BNC) + b0i_ref[...], 0.0)
    nu = (dot(au_lo_ref[...], wrel_u_lo_ref[...])
          + dot(au_hi_ref[...], wrel_u_hi_ref[...])
          + brel_u_ref[...] + dot(xu_ref[...], wro_iu_ref[...]))
    xu1_ref[...] = jnp.maximum(g0u_ref[...] * (nu * BNC) + b0u_ref[...], 0.0)


def _stage_c_body(a0_ref, a1_ref, xi1_ref, xi0_ref, wrel1_ref, wro1_ref,
                  brel1_ref, g1_ref, b1_ref, l1a_ref, l1b_ref, l1c_ref,
                  l1bias_ref, gl_ref, bl_ref, l2_ref, l2b_ref, out_ref):
    dot = functools.partial(jnp.dot, preferred_element_type=jnp.float32)
    agg = a0_ref[...] + a1_ref[...]
    ni = (dot(agg, wrel1_ref[...]) + brel1_ref[...]
          + dot(xi1_ref[...], wro1_ref[...]))
    xi2 = jnp.maximum(g1_ref[...] * (ni * BNC) + b1_ref[...], 0.0)
    h = (dot(xi0_ref[...], l1a_ref[...]) + dot(xi1_ref[...], l1b_ref[...])
         + dot(xi2, l1c_ref[...]) + l1bias_ref[...])
    h = jnp.maximum(gl_ref[...] * (h * BNC) + bl_ref[...], 0.0)
    out_ref[...] = dot(h, l2_ref[...]) + l2b_ref[...]


def _row_spec(d):
    return pl.BlockSpec((R, d), lambda i: (i, 0))


def _full_spec(a, b):
    return pl.BlockSpec((a, b), lambda i: (0, 0))


def kernel(x_item, x_user, edge_index_ub, edge_index_iu, y_emb, emb,
           W_rel0_ub, b_rel0_ub, W_root0_ub, W_rel0_iu, b_rel0_iu, W_root0_iu,
           bn0_item_g, bn0_item_b, bn0_user_g, bn0_user_b,
           W_rel1_ub, b_rel1_ub, W_root1_ub, W_rel1_iu, b_rel1_iu, W_root1_iu,
           bn1_item_g, bn1_item_b, bn1_user_g, bn1_user_b,
           lin1_W, lin1_b, bnl_g, bnl_b, lin2_W, lin2_b):
    f32 = jnp.float32
    row = lambda v: v.reshape(1, -1).astype(f32)

    xi0 = pl.pallas_call(
        _stage_a_body,
        grid=(NI // R,),
        in_specs=[_row_spec(DD), _row_spec(1), _full_spec(3, DD)],
        out_specs=[_row_spec(DD)],
        out_shape=[jax.ShapeDtypeStruct((NI, DD), f32)],
    )(x_item, y_emb.reshape(NI, 1).astype(jnp.int32), emb.astype(f32))[0]

    tot0 = NSUB * C0 * CH
    ub_src = _pad_idx(edge_index_ub[0], tot0).reshape(NSUB, C0, CH)
    ub_dst = _pad_dst(edge_index_ub[1], tot0).reshape(NSUB, C0, CH)
    iu_src = _pad_idx(edge_index_iu[0], tot0).reshape(NSUB, C0, CH)
    iu_dst = _pad_dst(edge_index_iu[1], tot0).reshape(NSUB, C0, CH)
    ai_lo, ai_hi, au_lo, au_hi = _agg_layer0(
        x_user[:, :HH], x_user[:, HH:], xi0[:, :HH], xi0[:, HH:],
        ub_src, ub_dst, iu_src, iu_dst)

    xi1, xu1 = pl.pallas_call(
        _stage_b_body,
        grid=(NI // R,),
        in_specs=[_row_spec(HH), _row_spec(HH), _row_spec(HH), _row_spec(HH),
                  _row_spec(DD), _row_spec(DD),
                  _full_spec(HH, HH), _full_spec(HH, HH),
                  _full_spec(HH, HH), _full_spec(HH, HH),
                  _full_spec(DD, HH), _full_spec(DD, HH),
                  _full_spec(1, HH), _full_spec(1, HH), _full_spec(1, HH),
                  _full_spec(1, HH), _full_spec(1, HH), _full_spec(1, HH)],
        out_specs=[_row_spec(HH), _row_spec(HH)],
        out_shape=[jax.ShapeDtypeStruct((NI, HH), f32),
                   jax.ShapeDtypeStruct((NU, HH), f32)],
    )(ai_lo, ai_hi, au_lo, au_hi, xi0, x_user,
      W_rel0_ub[:HH], W_rel0_ub[HH:], W_rel0_iu[:HH], W_rel0_iu[HH:],
      W_root0_ub, W_root0_iu,
      row(b_rel0_ub), row(b_rel0_iu), row(bn0_item_g), row(bn0_item_b),
      row(bn0_user_g), row(bn0_user_b))

    tot1 = 2 * NSUB * C1 * CH
    src4 = _pad_idx(edge_index_ub[0], tot1).reshape(2, NSUB, C1, CH)
    dst4 = _pad_dst(edge_index_ub[1], tot1).reshape(2, NSUB, C1, CH)
    agg1a, agg1b = _agg_layer1(xu1, src4, dst4)

    l2p = jnp.pad(lin2_W, ((0, 0), (0, 6)))
    l2bp = jnp.pad(lin2_b, (0, 6))
    out8 = pl.pallas_call(
        _stage_c_body,
        grid=(NI // R,),
        in_specs=[_row_spec(HH), _row_spec(HH), _row_spec(HH), _row_spec(DD),
                  _full_spec(HH, HH), _full_spec(HH, HH), _full_spec(1, HH),
                  _full_spec(1, HH), _full_spec(1, HH),
                  _full_spec(DD, HH), _full_spec(HH, HH), _full_spec(HH, HH),
                  _full_spec(1, HH), _full_spec(1, HH), _full_spec(1, HH),
                  _full_spec(HH, 8), _full_spec(1, 8)],
        out_specs=[_row_spec(8)],
        out_shape=[jax.ShapeDtypeStruct((NI, 8), f32)],
    )(agg1a, agg1b, xi1, xi0, W_rel1_ub, W_root1_ub, row(b_rel1_ub),
      row(bn1_item_g), row(bn1_item_b),
      lin1_W[:DD], lin1_W[DD:DD + HH], lin1_W[DD + HH:], row(lin1_b),
      row(bnl_g), row(bnl_b), l2p, row(l2bp))[0]

    return out8[:, :2]


# speedup vs baseline: 1.0191x; 1.0191x over previous
import functools
import math

import jax
import jax.numpy as jnp
from jax import lax
from jax.experimental import pallas as pl
from jax.experimental.pallas import tpu as pltpu
from jax.experimental.pallas import tpu_sc as plsc

NI = 10000
NU = 10000
DD = 256
HH = 128
EE = 320000
ROWS = 10240
DUMMY = NI
NSUB = 16
SLAB = ROWS // NSUB
CH = 64
G = 32
NB = 4
C0 = 320
C1 = 160
BNC = 1.0 / math.sqrt(1.0 + 1e-5)
R = 400


def _zero_fill(buf):
    zeros = jnp.zeros((16,), jnp.float32)

    def row(r, carry):
        def col(k, c2):
            buf[r, pl.ds(k * 16, 16)] = zeros
            return c2
        return lax.fori_loop(0, HH // 16, col, carry)

    lax.fori_loop(0, CH, row, 0)


def _agg_run(tab, src_slab, dst_slab, sidx, didx, bufs, acc,
             gsems, ssems, ngroups):
    def group(g, carry):
        pltpu.sync_copy(src_slab.at[pl.ds(g * G, G)], sidx)
        pltpu.sync_copy(dst_slab.at[pl.ds(g * G, G)], didx)
        for j in range(NB):
            pltpu.async_copy(tab.at[sidx.at[j]], bufs[j], gsems[j])

        def wave(w, c2):
            for j in range(NB):
                c = w * NB + j
                pltpu.make_async_copy(tab.at[sidx.at[c]],
                                      bufs[j], gsems[j]).wait()
                pltpu.async_copy(bufs[j], acc.at[didx.at[c]], ssems[j],
                                 add=True)

                @pl.when(w < G // NB - 1)
                def _(c=c, j=j):
                    pltpu.make_async_copy(bufs[j], acc.at[didx.at[c]],
                                          ssems[j]).wait()
                    pltpu.async_copy(tab.at[sidx.at[c + NB]],
                                     bufs[j], gsems[j])

            return c2

        lax.fori_loop(0, G // NB, wave, carry)
        for j in range(NB):
            pltpu.make_async_copy(bufs[j], acc.at[didx.at[G - NB + j]],
                                  ssems[j]).wait()
        return carry

    lax.fori_loop(0, ngroups, group, 0)


def _acc_zero(buf, acc, sid):
    _zero_fill(buf)
    for j in range(SLAB // CH):
        pltpu.sync_copy(buf, acc.at[pl.ds(sid * SLAB + j * CH, CH)])


def _writeback(acc, out, sid):
    pltpu.sync_copy(acc.at[pl.ds(sid * SLAB, SLAB)],
                    out.at[pl.ds(sid * SLAB, SLAB)])


_SC_MESH = plsc.VectorSubcoreMesh(core_axis_name="c", subcore_axis_name="s")

_AGG_SCRATCH = (
    [pltpu.VMEM((G, CH), jnp.int32)] * 2
    + [pltpu.VMEM((CH, HH), jnp.float32)] * NB
    + [pltpu.VMEM_SHARED((ROWS, HH), jnp.float32)]
    + [pltpu.SemaphoreType.DMA] * (2 * NB)
)


@functools.partial(
    pl.kernel,
    mesh=_SC_MESH,
    out_type=[jax.ShapeDtypeStruct((ROWS, HH), jnp.float32)] * 4,
    scratch_types=_AGG_SCRATCH,
)
def _agg_layer0(xu_lo, xu_hi, xi_lo, xi_hi, ub_src, ub_dst, iu_src, iu_dst,
                oi_lo, oi_hi, ou_lo, ou_hi,
                sidx, didx, b0, b1, b2, b3, acc,
                g0, g1, g2, g3, s0, s1, s2, s3):
    cid = lax.axis_index("c")
    sid = lax.axis_index("s")
    bufs, gsems, ssems = [b0, b1, b2, b3], [g0, g1, g2, g3], [s0, s1, s2, s3]

    def one_pass(tab_lo, tab_hi, src, dst, out_lo, out_hi):
        _acc_zero(b0, acc, sid)
        plsc.subcore_barrier()

        @pl.when(cid == 0)
        def _():
            _agg_run(tab_lo, src.at[sid], dst.at[sid], sidx, didx,
                     bufs, acc, gsems, ssems, C0 // G)

        @pl.when(cid != 0)
        def _():
            _agg_run(tab_hi, src.at[sid], dst.at[sid], sidx, didx,
                     bufs, acc, gsems, ssems, C0 // G)

        plsc.subcore_barrier()

        @pl.when(cid == 0)
        def _():
            _writeback(acc, out_lo, sid)

        @pl.when(cid != 0)
        def _():
            _writeback(acc, out_hi, sid)

        plsc.subcore_barrier()

    one_pass(xu_lo, xu_hi, ub_src, ub_dst, oi_lo, oi_hi)
    one_pass(xi_lo, xi_hi, iu_src, iu_dst, ou_lo, ou_hi)


@functools.partial(
    pl.kernel,
    mesh=_SC_MESH,
    out_type=[jax.ShapeDtypeStruct((ROWS, HH), jnp.float32)] * 2,
    scratch_types=_AGG_SCRATCH,
)
def _agg_layer1(tab, src4, dst4, out_a, out_b, sidx, didx,
                b0, b1, b2, b3, acc, g0, g1, g2, g3, s0, s1, s2, s3):
    cid = lax.axis_index("c")
    sid = lax.axis_index("s")
    _acc_zero(b0, acc, sid)
    plsc.subcore_barrier()

    _agg_run(tab, src4.at[cid, sid], dst4.at[cid, sid], sidx, didx,
             [b0, b1, b2, b3], acc, [g0, g1, g2, g3], [s0, s1, s2, s3],
             C1 // G)

    plsc.subcore_barrier()

    @pl.when(cid == 0)
    def _():
        _writeback(acc, out_a, sid)

    @pl.when(cid != 0)
    def _():
        _writeback(acc, out_b, sid)


def _pad_idx(idx, total):
    pad = jnp.arange(total - EE, dtype=jnp.int32) % NI
    return jnp.concatenate([idx.astype(jnp.int32), pad])


def _pad_dst(idx, total):
    pad = jnp.arange(total - EE, dtype=jnp.int32) % (ROWS - NI) + DUMMY
    return jnp.concatenate([idx.astype(jnp.int32), pad])


def _stage_a_body(xi_ref, y_ref, emb_ref, xi0_ref):
    y = y_ref[...]
    e0 = emb_ref[0:1, :]
    e1 = emb_ref[1:2, :]
    sel0 = jnp.where(y == 0, 1.0, 0.0)
    sel1 = jnp.where(y == 1, 1.0, 0.0)
    xi0_ref[...] = xi_ref[...] + sel0 * e0 + sel1 * e1


def _stage_b_body(ai_lo_ref, ai_hi_ref, au_lo_ref, au_hi_ref, xi0_ref, xu_ref,
                  wrel_i_lo_ref, wrel_i_hi_ref, wrel_u_lo_ref, wrel_u_hi_ref,
                  wro_ub_ref, wro_iu_ref,
                  brel_i_ref, brel_u_ref, g0i_ref, b0i_ref, g0u_ref, b0u_ref,
                  xi1_ref, xu1_ref):
    dot = functools.partial(jnp.dot, preferred_element_type=jnp.float32)
    ni = (dot(ai_lo_ref[...], wrel_i_lo_ref[...])
          + dot(ai_hi_ref[...], wrel_i_hi_ref[...])
          + brel_i_ref[...] + dot(xi0_ref[...], wro_ub_ref[...]))
    xi1_ref[...] = jnp.maximum(g0i_ref[...] * (ni * BNC) + b0i_ref[...], 0.0)
    nu = (dot(au_lo_ref[...], wrel_u_lo_ref[...])
          + dot(au_hi_ref[...], wrel_u_hi_ref[...])
          + brel_u_ref[...] + dot(xu_ref[...], wro_iu_ref[...]))
    xu1_ref[...] = jnp.maximum(g0u_ref[...] * (nu * BNC) + b0u_ref[...], 0.0)


def _stage_c_body(a0_ref, a1_ref, xi1_ref, xi0_ref, wrel1_ref, wro1_ref,
                  brel1_ref, g1_ref, b1_ref, l1a_ref, l1b_ref, l1c_ref,
                  l1bias_ref, gl_ref, bl_ref, l2_ref, l2b_ref, out_ref):
    dot = functools.partial(jnp.dot, preferred_element_type=jnp.float32)
    agg = a0_ref[...] + a1_ref[...]
    ni = (dot(agg, wrel1_ref[...]) + brel1_ref[...]
          + dot(xi1_ref[...], wro1_ref[...]))
    xi2 = jnp.maximum(g1_ref[...] * (ni * BNC) + b1_ref[...], 0.0)
    h = (dot(xi0_ref[...], l1a_ref[...]) + dot(xi1_ref[...], l1b_ref[...])
         + dot(xi2, l1c_ref[...]) + l1bias_ref[...])
    h = jnp.maximum(gl_ref[...] * (h * BNC) + bl_ref[...], 0.0)
    out_ref[...] = dot(h, l2_ref[...]) + l2b_ref[...]


def _row_spec(d):
    return pl.BlockSpec((R, d), lambda i: (i, 0))


def _full_spec(a, b):
    return pl.BlockSpec((a, b), lambda i: (0, 0))


def kernel(x_item, x_user, edge_index_ub, edge_index_iu, y_emb, emb,
           W_rel0_ub, b_rel0_ub, W_root0_ub, W_rel0_iu, b_rel0_iu, W_root0_iu,
           bn0_item_g, bn0_item_b, bn0_user_g, bn0_user_b,
           W_rel1_ub, b_rel1_ub, W_root1_ub, W_rel1_iu, b_rel1_iu, W_root1_iu,
           bn1_item_g, bn1_item_b, bn1_user_g, bn1_user_b,
           lin1_W, lin1_b, bnl_g, bnl_b, lin2_W, lin2_b):
    f32 = jnp.float32
    row = lambda v: v.reshape(1, -1).astype(f32)

    xi0 = pl.pallas_call(
        _stage_a_body,
        grid=(NI // R,),
        in_specs=[_row_spec(DD), _row_spec(1), _full_spec(3, DD)],
        out_specs=[_row_spec(DD)],
        out_shape=[jax.ShapeDtypeStruct((NI, DD), f32)],
    )(x_item, y_emb.reshape(NI, 1).astype(jnp.int32), emb.astype(f32))[0]

    tot0 = NSUB * C0 * CH
    ub_src = _pad_idx(edge_index_ub[0], tot0).reshape(NSUB, C0, CH)
    ub_dst = _pad_dst(edge_index_ub[1], tot0).reshape(NSUB, C0, CH)
    iu_src = _pad_idx(edge_index_iu[0], tot0).reshape(NSUB, C0, CH)
    iu_dst = _pad_dst(edge_index_iu[1], tot0).reshape(NSUB, C0, CH)
    ai_lo, ai_hi, au_lo, au_hi = _agg_layer0(
        x_user[:, :HH], x_user[:, HH:], xi0[:, :HH], xi0[:, HH:],
        ub_src, ub_dst, iu_src, iu_dst)

    xi1, xu1 = pl.pallas_call(
        _stage_b_body,
        grid=(NI // R,),
        in_specs=[_row_spec(HH), _row_spec(HH), _row_spec(HH), _row_spec(HH),
                  _row_spec(DD), _row_spec(DD),
                  _full_spec(HH, HH), _full_spec(HH, HH),
                  _full_spec(HH, HH), _full_spec(HH, HH),
                  _full_spec(DD, HH), _full_spec(DD, HH),
                  _full_spec(1, HH), _full_spec(1, HH), _full_spec(1, HH),
                  _full_spec(1, HH), _full_spec(1, HH), _full_spec(1, HH)],
        out_specs=[_row_spec(HH), _row_spec(HH)],
        out_shape=[jax.ShapeDtypeStruct((NI, HH), f32),
                   jax.ShapeDtypeStruct((NU, HH), f32)],
    )(ai_lo, ai_hi, au_lo, au_hi, xi0, x_user,
      W_rel0_ub[:HH], W_rel0_ub[HH:], W_rel0_iu[:HH], W_rel0_iu[HH:],
      W_root0_ub, W_root0_iu,
      row(b_rel0_ub), row(b_rel0_iu), row(bn0_item_g), row(bn0_item_b),
      row(bn0_user_g), row(bn0_user_b))

    tot1 = 2 * NSUB * C1 * CH
    src4 = _pad_idx(edge_index_ub[0], tot1).reshape(2, NSUB, C1, CH)
    dst4 = _pad_dst(edge_index_ub[1], tot1).reshape(2, NSUB, C1, CH)
    agg1a, agg1b = _agg_layer1(xu1, src4, dst4)

    l2p = jnp.pad(lin2_W, ((0, 0), (0, 6)))
    l2bp = jnp.pad(lin2_b, (0, 6))
    out8 = pl.pallas_call(
        _stage_c_body,
        grid=(NI // R,),
        in_specs=[_row_spec(HH), _row_spec(HH), _row_spec(HH), _row_spec(DD),
                  _full_spec(HH, HH), _full_spec(HH, HH), _full_spec(1, HH),
                  _full_spec(1, HH), _full_spec(1, HH),
                  _full_spec(DD, HH), _full_spec(HH, HH), _full_spec(HH, HH),
                  _full_spec(1, HH), _full_spec(1, HH), _full_spec(1, HH),
                  _full_spec(HH, 8), _full_spec(1, 8)],
        out_specs=[_row_spec(8)],
        out_shape=[jax.ShapeDtypeStruct((NI, 8), f32)],
    )(agg1a, agg1b, xi1, xi0, W_rel1_ub, W_root1_ub, row(b_rel1_ub),
      row(bn1_item_g), row(bn1_item_b),
      lin1_W[:DD], lin1_W[DD:DD + HH], lin1_W[DD + HH:], row(lin1_b),
      row(bnl_g), row(bnl_b), l2p, row(l2bp))[0]

    return out8[:, :2]
